# baseline (speedup 1.0000x reference)
import functools

import jax
import jax.numpy as jnp
from jax import lax
from jax.experimental import pallas as pl
from jax.experimental.pallas import tpu as pltpu

N = 1_000_000
NP = 1_003_520
C = 6272
G = NP // C
PAD_CODE = (1 << 30) - 1


def _expand3(v):
    x = v
    x = (x | (x << 16)) & 0x30000FF
    x = (x | (x << 8)) & 0x300F00F
    x = (x | (x << 4)) & 0x30C30C3
    x = (x | (x << 2)) & 0x9249249
    return x


def _morton_body(pc_ref, codes_ref, mm_ref):
    ph = pl.program_id(0)
    g = pl.program_id(1)

    @pl.when(ph == 0)
    def _():
        blk = pc_ref[...]
        bmin = jnp.min(blk, axis=1, keepdims=True)
        bmax = jnp.max(blk, axis=1, keepdims=True)
        prev_min = jnp.where(g == 0, jnp.full_like(bmin, jnp.inf), mm_ref[:, 0:1])
        prev_max = jnp.where(g == 0, jnp.full_like(bmax, -jnp.inf), mm_ref[:, 1:2])
        mm_ref[:, 0:1] = jnp.minimum(prev_min, bmin)
        mm_ref[:, 1:2] = jnp.maximum(prev_max, bmax)

    @pl.when(ph == 1)
    def _():
        blk = pc_ref[...]
        bmin = mm_ref[:, 0:1]
        bmax = mm_ref[:, 1:2]
        scale = jnp.float32(1023) / (bmax - bmin + jnp.float32(1e-7))
        q = jnp.floor((blk - bmin) * scale).astype(jnp.int32)
        q = jnp.minimum(q, 1023)
        e = _expand3(q)
        code = (e[0:1, :] << 2) | (e[1:2, :] << 1) | e[2:3, :]
        pos = g * C + lax.broadcasted_iota(jnp.int32, (1, C), 1)
        codes_ref[...] = jnp.where(pos < N, code, PAD_CODE)


def _morton_codes_padded(pcp):
    return pl.pallas_call(
        _morton_body,
        grid=(2, G),
        in_specs=[pl.BlockSpec((3, C), lambda ph, g: (0, g))],
        out_specs=pl.BlockSpec((1, C), lambda ph, g: (0, g)),
        out_shape=jax.ShapeDtypeStruct((1, NP), jnp.int32),
        scratch_shapes=[pltpu.VMEM((3, 2), jnp.float32)],
    )(pcp)


def kernel(pointcloud, color):
    pct = pointcloud.T
    pcp = jnp.pad(pct, ((0, 0), (0, NP - N)))
    codes = _morton_codes_padded(pcp)[0, :N].astype(jnp.uint32)
    order = jnp.argsort(codes)
    return (
        jnp.take(pointcloud, order, axis=0),
        jnp.take(color, order, axis=0),
        jnp.take(codes, order),
    )
